# Initial kernel scaffold; baseline (speedup 1.0000x reference)
import jax
import jax.numpy as jnp
from jax.experimental import pallas as pl


def kernel(inputs, hidden_state, edge_index, W_gcn, b_gcn, W_ih, W_hh, b_ih, b_hh, W_fc2, b_fc2):
    raise NotImplementedError("write your pallas kernel here")


# speedup vs baseline: 34.1364x; 34.1364x over previous
import functools

import jax
import jax.numpy as jnp
from jax import lax
from jax.experimental import pallas as pl
from jax.experimental.pallas import tpu as pltpu
from jax.experimental.pallas import tpu_sc as plsc

N_NODES = 10000
N_PAD = 10240
E_TOT = 320000
D_FEAT = 128
HID = 128
ACT = 16
NC, NS = 2, 16
NW = NC * NS
CHUNK = 128
NCHUNK = 79
E_PAD = NW * NCHUNK * CHUNK
NPAIR = (NCHUNK - 1) // 2
ROWS_PT = N_PAD // NS

_mesh = plsc.VectorSubcoreMesh(core_axis_name="c", subcore_axis_name="s")


@functools.partial(
    pl.kernel,
    out_type=jax.ShapeDtypeStruct((NC, N_PAD), jnp.float32),
    mesh=_mesh,
    scratch_types=[
        pltpu.VMEM((NCHUNK, CHUNK), jnp.int32),
        pltpu.VMEM((CHUNK,), jnp.float32),
        pltpu.VMEM((ROWS_PT,), jnp.float32),
        pltpu.VMEM_SHARED((N_PAD,), jnp.float32),
        pltpu.SemaphoreType.DMA,
    ],
)
def _deg_kernel(dst_hbm, deg_out, idx_v, ones_v, zero_v, deg_sh, sem):
    c = lax.axis_index("c")
    s = lax.axis_index("s")
    wid = s * NC + c
    row0 = s * ROWS_PT

    for i in range(CHUNK // 16):
        ones_v[pl.ds(i * 16, 16)] = jnp.ones((16,), jnp.float32)
    for i in range(ROWS_PT // 16):
        zero_v[pl.ds(i * 16, 16)] = jnp.zeros((16,), jnp.float32)
    pltpu.sync_copy(zero_v, deg_sh.at[pl.ds(row0, ROWS_PT)])
    pltpu.sync_copy(dst_hbm.at[wid], idx_v)
    plsc.subcore_barrier()

    def body(j, carry):
        pltpu.sync_copy(ones_v, deg_sh.at[idx_v.at[j]], add=True)
        return carry

    lax.fori_loop(0, NCHUNK, body, 0)
    plsc.subcore_barrier()
    pltpu.sync_copy(deg_sh.at[pl.ds(row0, ROWS_PT)],
                    deg_out.at[c, pl.ds(row0, ROWS_PT)])


@functools.partial(
    pl.kernel,
    out_type=jax.ShapeDtypeStruct((NC, N_PAD, HID), jnp.float32),
    mesh=_mesh,
    scratch_types=[
        pltpu.VMEM((NCHUNK, CHUNK), jnp.int32),
        pltpu.VMEM((1, CHUNK), jnp.int32),
        pltpu.VMEM((1, CHUNK), jnp.int32),
        pltpu.VMEM((CHUNK, HID), jnp.float32),
        pltpu.VMEM((CHUNK, HID), jnp.float32),
        pltpu.VMEM_SHARED((N_PAD, HID), jnp.float32),
        pltpu.SemaphoreType.DMA,
        pltpu.SemaphoreType.DMA,
        pltpu.SemaphoreType.DMA,
        pltpu.SemaphoreType.DMA,
    ],
)
def _msg_kernel(xw_hbm, src_hbm, dst_hbm, acc_out,
                src_v, didx0, didx1, buf0, buf1, acc_sh,
                sem0, sem1, semi0, semi1):
    c = lax.axis_index("c")
    s = lax.axis_index("s")
    wid = s * NC + c
    row0 = s * ROWS_PT

    @pl.when(c == 0)
    def _():
        pltpu.sync_copy(xw_hbm.at[pl.ds(row0, ROWS_PT)],
                        acc_sh.at[pl.ds(row0, ROWS_PT)])

    @pl.when(c == 1)
    def _():
        def zb(i, carry):
            for l in range(HID // 16):
                buf0[i, pl.ds(l * 16, 16)] = jnp.zeros((16,), jnp.float32)
            return carry
        lax.fori_loop(0, CHUNK, zb, 0)

        def cp(i, carry):
            pltpu.sync_copy(buf0, acc_sh.at[pl.ds(row0 + i * CHUNK, CHUNK)])
            return carry
        lax.fori_loop(0, ROWS_PT // CHUNK, cp, 0)

    pltpu.sync_copy(src_hbm.at[wid], src_v)
    plsc.subcore_barrier()

    pltpu.async_copy(xw_hbm.at[src_v.at[0]], buf0, sem0)
    pltpu.async_copy(dst_hbm.at[wid, pl.ds(0, 1)], didx0, semi0)

    def pair(p, carry):
        j0 = 2 * p
        pltpu.async_copy(xw_hbm.at[src_v.at[j0 + 1]], buf1, sem1)
        pltpu.async_copy(dst_hbm.at[wid, pl.ds(j0 + 1, 1)], didx1, semi1)
        pltpu.make_async_copy(xw_hbm.at[src_v.at[j0]], buf0, sem0).wait()
        pltpu.make_async_copy(dst_hbm.at[wid, pl.ds(j0, 1)], didx0, semi0).wait()
        pltpu.sync_copy(buf0, acc_sh.at[didx0.at[0]], add=True)
        pltpu.async_copy(xw_hbm.at[src_v.at[j0 + 2]], buf0, sem0)
        pltpu.async_copy(dst_hbm.at[wid, pl.ds(j0 + 2, 1)], didx0, semi0)
        pltpu.make_async_copy(xw_hbm.at[src_v.at[j0 + 1]], buf1, sem1).wait()
        pltpu.make_async_copy(dst_hbm.at[wid, pl.ds(j0 + 1, 1)], didx1, semi1).wait()
        pltpu.sync_copy(buf1, acc_sh.at[didx1.at[0]], add=True)
        return carry

    lax.fori_loop(0, NPAIR, pair, 0)
    pltpu.make_async_copy(xw_hbm.at[src_v.at[NCHUNK - 1]], buf0, sem0).wait()
    pltpu.make_async_copy(dst_hbm.at[wid, pl.ds(NCHUNK - 1, 1)], didx0, semi0).wait()
    pltpu.sync_copy(buf0, acc_sh.at[didx0.at[0]], add=True)

    plsc.subcore_barrier()
    pltpu.sync_copy(acc_sh.at[pl.ds(row0, ROWS_PT)],
                    acc_out.at[c, pl.ds(row0, ROWS_PT)])


_BLK = 512
_GRID = N_PAD // _BLK


def _xw_body(x_ref, w_ref, degt_ref, out_ref):
    deg = degt_ref[...]
    dinv = lax.rsqrt(deg[:, 0] + deg[:, 1] + 1.0)
    xw = jnp.dot(x_ref[...], w_ref[...], preferred_element_type=jnp.float32)
    out_ref[...] = xw * dinv[:, None]


def _run_xw(x_pad, w, degt):
    return pl.pallas_call(
        _xw_body,
        grid=(_GRID,),
        in_specs=[
            pl.BlockSpec((_BLK, D_FEAT), lambda i: (i, 0)),
            pl.BlockSpec((D_FEAT, HID), lambda i: (0, 0)),
            pl.BlockSpec((_BLK, 2), lambda i: (i, 0)),
        ],
        out_specs=pl.BlockSpec((_BLK, HID), lambda i: (i, 0)),
        out_shape=jax.ShapeDtypeStruct((N_PAD, HID), jnp.float32),
    )(x_pad, w, degt)


def _head_body(acc0_ref, acc1_ref, degt_ref, h_ref, bgcn_ref,
               wiht_ref, whht_ref, bih_ref, bhh_ref, wfc2t_ref, bfc2_ref,
               q_ref, hout_ref):
    deg = degt_ref[...]
    dinv = lax.rsqrt(deg[:, 0] + deg[:, 1] + 1.0)
    x = jnp.maximum((acc0_ref[...] + acc1_ref[...]) * dinv[:, None]
                    + bgcn_ref[...], 0.0)
    h = h_ref[...]
    gi = jnp.dot(x, wiht_ref[...], preferred_element_type=jnp.float32) \
        + bih_ref[...]
    gh = jnp.dot(h, whht_ref[...], preferred_element_type=jnp.float32) \
        + bhh_ref[...]
    r = jax.nn.sigmoid(gi[:, :HID] + gh[:, :HID])
    z = jax.nn.sigmoid(gi[:, HID:2 * HID] + gh[:, HID:2 * HID])
    n = jnp.tanh(gi[:, 2 * HID:] + r * gh[:, 2 * HID:])
    hn = (1.0 - z) * n + z * h
    hout_ref[...] = hn
    q_ref[...] = jnp.dot(hn, wfc2t_ref[...],
                         preferred_element_type=jnp.float32) + bfc2_ref[...]


def _run_head(acc0, acc1, degt, h_pad, bgcn, wiht, whht, bih, bhh, wfc2t, bfc2):
    return pl.pallas_call(
        _head_body,
        grid=(_GRID,),
        in_specs=[
            pl.BlockSpec((_BLK, HID), lambda i: (i, 0)),
            pl.BlockSpec((_BLK, HID), lambda i: (i, 0)),
            pl.BlockSpec((_BLK, 2), lambda i: (i, 0)),
            pl.BlockSpec((_BLK, HID), lambda i: (i, 0)),
            pl.BlockSpec((1, HID), lambda i: (0, 0)),
            pl.BlockSpec((HID, 3 * HID), lambda i: (0, 0)),
            pl.BlockSpec((HID, 3 * HID), lambda i: (0, 0)),
            pl.BlockSpec((1, 3 * HID), lambda i: (0, 0)),
            pl.BlockSpec((1, 3 * HID), lambda i: (0, 0)),
            pl.BlockSpec((HID, ACT), lambda i: (0, 0)),
            pl.BlockSpec((1, ACT), lambda i: (0, 0)),
        ],
        out_specs=[
            pl.BlockSpec((_BLK, ACT), lambda i: (i, 0)),
            pl.BlockSpec((_BLK, HID), lambda i: (i, 0)),
        ],
        out_shape=[
            jax.ShapeDtypeStruct((N_PAD, ACT), jnp.float32),
            jax.ShapeDtypeStruct((N_PAD, HID), jnp.float32),
        ],
    )(acc0, acc1, degt, h_pad, bgcn, wiht, whht, bih, bhh, wfc2t, bfc2)


def kernel(inputs, hidden_state, edge_index, W_gcn, b_gcn,
           W_ih, W_hh, b_ih, b_hh, W_fc2, b_fc2):
    pad_idx = N_NODES + jnp.arange(E_PAD - E_TOT, dtype=jnp.int32) \
        % (N_PAD - N_NODES)
    src_slab = jnp.concatenate([edge_index[0], pad_idx]) \
        .reshape(NW, NCHUNK, CHUNK)
    dst_slab = jnp.concatenate([edge_index[1], pad_idx]) \
        .reshape(NW, NCHUNK, CHUNK)

    deg_parts = _deg_kernel(dst_slab)
    degt = deg_parts.T

    x_pad = jnp.pad(inputs, ((0, N_PAD - N_NODES), (0, 0)))
    xw = _run_xw(x_pad, W_gcn, degt)

    acc = _msg_kernel(xw, src_slab, dst_slab)

    h_pad = jnp.pad(hidden_state.reshape(N_NODES, HID),
                    ((0, N_PAD - N_NODES), (0, 0)))
    q_pad, h_new = _run_head(
        acc[0], acc[1], degt, h_pad,
        b_gcn.reshape(1, HID), W_ih.T, W_hh.T,
        b_ih.reshape(1, 3 * HID), b_hh.reshape(1, 3 * HID),
        W_fc2.T, b_fc2.reshape(1, ACT))
    return (q_pad[:N_NODES], h_new[:N_NODES])
